# baseline (speedup 1.0000x reference)
import functools

import jax
import jax.numpy as jnp
from jax.experimental import pallas as pl
from jax.experimental.pallas import tpu as pltpu

_V = 1.0
_F = 128


def _layer1_kernel(adj_ref, x_ref, w1_ref, w2_ref, snext_ref, adjb_ref):
    adjb = adj_ref[...].astype(jnp.bfloat16)
    adjb_ref[...] = adjb
    s1 = jnp.dot(x_ref[...], w1_ref[...],
                 preferred_element_type=jnp.float32).astype(jnp.bfloat16)
    h = jnp.dot(adjb, s1, preferred_element_type=jnp.float32)
    h = jnp.maximum(h, 0.0)
    snext_ref[...] = jnp.dot(h, w2_ref[...],
                             preferred_element_type=jnp.float32
                             ).astype(snext_ref.dtype)


def _dotT(a, b):
    return jax.lax.dot_general(a, b, (((1,), (1,)), ((), ())),
                               preferred_element_type=jnp.float32)


def _mega_kernel(adjb_ref, s2_ref, w_ref, c_ref,
                 z_ref, zc_ref, zhat_ref, zadj_ref, q_ref,
                 sc0, sc1, zscr, zcscr, *, bm, k, e):
    s = pl.program_id(0)
    i = pl.program_id(1)
    n = adjb_ref.shape[1]
    row = pl.ds(i * bm, bm)
    a = adjb_ref[...]

    @pl.when((s == 0) & (i == 0))
    def _():
        s2 = s2_ref[...]
        sc0[...] = jnp.concatenate(
            [s2, jnp.zeros((n, _F - s2.shape[1]), jnp.bfloat16)], axis=1)

    @pl.when(s == 0)
    def _():
        h = jnp.maximum(jnp.dot(a, sc0[...],
                                preferred_element_type=jnp.float32), 0.0)
        sc1[row, :] = jnp.dot(h, w_ref[0],
                              preferred_element_type=jnp.float32
                              ).astype(jnp.bfloat16)

    @pl.when(s == 1)
    def _():
        h = jnp.dot(a, sc1[...], preferred_element_type=jnp.float32)
        zscr[row, :] = h[:, :e]
        sc0[row, :] = jnp.dot(h, w_ref[0],
                              preferred_element_type=jnp.float32
                              ).astype(jnp.bfloat16)

    @pl.when(s == 2)
    def _():
        h = jnp.maximum(jnp.dot(a, sc0[...],
                                preferred_element_type=jnp.float32), 0.0)
        zcscr[row, :] = h[:, :k]
        sc1[row, :] = jnp.dot(h, w_ref[0],
                              preferred_element_type=jnp.float32
                              ).astype(jnp.bfloat16)

    @pl.when(s == 3)
    def _():
        h = jnp.maximum(jnp.dot(a, sc1[...],
                                preferred_element_type=jnp.float32), 0.0)
        sc0[row, :] = jnp.dot(h, w_ref[0],
                              preferred_element_type=jnp.float32
                              ).astype(jnp.bfloat16)

    @pl.when(s == 4)
    def _():
        h = jnp.maximum(jnp.dot(a, sc0[...],
                                preferred_element_type=jnp.float32), 0.0)
        zhat_ref[...] = h
        zb = zscr[row, :]
        z_ref[...] = zb
        zc_ref[...] = zcscr[row, :]
        zadj_ref[...] = jax.nn.sigmoid(_dotT(zb, zscr[...]))
        c = c_ref[...]
        d2 = (jnp.sum(zb * zb, axis=1, keepdims=True)
              + jnp.sum(c * c, axis=1)[None, :] - 2.0 * _dotT(zb, c))
        qn = 1.0 / (1.0 + d2 / _V)
        qn = qn ** ((_V + 1.0) / 2.0)
        qn = qn[:, :k]
        q_ref[...] = qn / jnp.sum(qn, axis=1, keepdims=True)


def _block_m(n, target):
    for bm in (1000, 400, 200, 8):
        if bm <= target and n % bm == 0:
            return bm
    return n


def kernel(x, adj, W1, W2, W3, Wc, W4, W5, W6, cluster_layer):
    n = adj.shape[0]
    k, e = cluster_layer.shape
    kpad = max(8, -(-k // 8) * 8)
    c_pad = jnp.zeros((kpad, e), jnp.float32).at[:k].set(cluster_layer)
    f32, bf16 = jnp.float32, jnp.bfloat16

    bm1 = _block_m(n, 200)
    s2, adjb = pl.pallas_call(
        _layer1_kernel,
        grid=(n // bm1,),
        in_specs=[pl.BlockSpec((bm1, n), lambda i: (i, 0)),
                  pl.BlockSpec(x.shape, lambda i: (0, 0)),
                  pl.BlockSpec(W1.shape, lambda i: (0, 0)),
                  pl.BlockSpec(W2.shape, lambda i: (0, 0))],
        out_specs=[pl.BlockSpec((bm1, W2.shape[1]), lambda i: (i, 0)),
                   pl.BlockSpec((bm1, n), lambda i: (i, 0))],
        out_shape=[jax.ShapeDtypeStruct((n, W2.shape[1]), bf16),
                   jax.ShapeDtypeStruct((n, n), bf16)])(adj, x, W1, W2)

    w_cat = jnp.concatenate([Wc, W4], axis=1)
    wpad = jnp.zeros((5, _F, _F), f32)
    wpad = wpad.at[0, :W3.shape[0], :W3.shape[1]].set(W3)
    wpad = wpad.at[1, :w_cat.shape[0], :w_cat.shape[1]].set(w_cat)
    wpad = wpad.at[2, k:k + W5.shape[0], :W5.shape[1]].set(W5)
    wpad = wpad.at[3, :W6.shape[0], :W6.shape[1]].set(W6)

    bm = _block_m(n, 200)
    nb = n // bm
    last = lambda s, i: (jnp.where(s == 4, i, 0), 0)
    z, z_cluster, z_hat, z_adj, q = pl.pallas_call(
        functools.partial(_mega_kernel, bm=bm, k=k, e=e),
        grid=(5, nb),
        in_specs=[
            pl.BlockSpec((bm, n), lambda s, i: (i, 0)),
            pl.BlockSpec((n, W2.shape[1]), lambda s, i: (0, 0)),
            pl.BlockSpec((1, _F, _F), lambda s, i: (s, 0, 0)),
            pl.BlockSpec((kpad, e), lambda s, i: (0, 0)),
        ],
        out_specs=[
            pl.BlockSpec((bm, e), last),
            pl.BlockSpec((bm, k), last),
            pl.BlockSpec((bm, _F), last),
            pl.BlockSpec((bm, n), last),
            pl.BlockSpec((bm, k), last),
        ],
        out_shape=[
            jax.ShapeDtypeStruct((n, e), f32),
            jax.ShapeDtypeStruct((n, k), f32),
            jax.ShapeDtypeStruct((n, _F), f32),
            jax.ShapeDtypeStruct((n, n), f32),
            jax.ShapeDtypeStruct((n, k), f32),
        ],
        scratch_shapes=[
            pltpu.VMEM((n, _F), bf16),
            pltpu.VMEM((n, _F), bf16),
            pltpu.VMEM((n, e), f32),
            pltpu.VMEM((n, k), f32),
        ])(adjb, s2, wpad, c_pad)

    return (z_hat, z_adj, z, z_cluster, q)
